# baseline (speedup 1.0000x reference)
import jax
import jax.numpy as jnp
from jax.experimental import pallas as pl

Bv, Nv, Cv = 2, 2048, 64
NSAMPLE = 16
G = 4


def _conv1d(x, W, b):
    return jnp.einsum('oc,bcn->bon', W, x) + b[None, :, None]


def _conv2d(x, W, b):
    return jnp.einsum('oc,bcns->bons', W, x) + b[None, :, None, None]


def _gn(x, gamma, beta):
    B_, C_, N_, S_ = x.shape
    xr = x.reshape(B_, G, C_ // G, N_, S_)
    m = xr.mean(axis=(2, 3, 4), keepdims=True)
    v = xr.var(axis=(2, 3, 4), keepdims=True)
    xr = (xr - m) / jnp.sqrt(v + 1e-5)
    x = xr.reshape(B_, C_, N_, S_)
    return x * gamma[None, :, None, None] + beta[None, :, None, None]


def _leaky(x):
    return jnp.where(x >= 0, x, 0.1 * x)


def _group(x, idx):
    return jax.vmap(lambda xb, ib: xb[:, ib])(x, idx)


def _cross(xyz1, xyz2, p1, p2, posW, posb, mlps, g0, b0):
    dists = jnp.sqrt(jnp.sum((xyz1[:, :, :, None] - xyz2[:, :, None, :]) ** 2, axis=1))
    _, knn_idx = jax.lax.top_k(-dists, NSAMPLE)
    nxyz = _group(xyz2, knn_idx)
    dxyz = nxyz - xyz1[:, :, :, None]
    gp2 = _group(p2, knn_idx)
    gp1 = jnp.broadcast_to(p1[:, :, :, None], gp2.shape)
    pos = _conv2d(dxyz, posW, posb)
    h = _leaky(_gn(gp2 + gp1 + pos, g0, b0))
    for (W, b, gg, bb) in mlps:
        h = _leaky(_gn(_conv2d(h, W, b), gg, bb))
    return jnp.max(h, axis=3)


def _conv1d_pallas(x, W, b):
    def body(w_ref, b_ref, x_ref, o_ref):
        for i in range(x_ref.shape[0]):
            o_ref[i] = (
                jax.lax.dot(w_ref[...], x_ref[i],
                            preferred_element_type=jnp.float32)
                + b_ref[...][:, None]
            )

    return pl.pallas_call(
        body,
        out_shape=jax.ShapeDtypeStruct((x.shape[0], W.shape[0], x.shape[2]),
                                       jnp.float32),
    )(W, b, x)


def kernel(pc1, pc2, feat1, feat2, W_t11, b_t11, W_t22, b_t22, W_pos1, b_pos1, gamma_bn1, beta_bn1, Wm1_1, bm1_1, gm1_1, btm1_1, Wm1_2, bm1_2, gm1_2, btm1_2, W_t1, b_t1, W_t2, b_t2, W_pos2, b_pos2, gamma_bn2, beta_bn2, Wm2_1, bm2_1, gm2_1, btm2_1):
    mlps1 = [(Wm1_1, bm1_1, gm1_1, btm1_1), (Wm1_2, bm1_2, gm1_2, btm1_2)]
    mlps2 = [(Wm2_1, bm2_1, gm2_1, btm2_1)]
    f1 = _conv1d_pallas(feat1, W_t11, b_t11)
    f2 = _conv1d_pallas(feat2, W_t22, b_t22)
    feat1_new = _cross(pc1, pc2, f1, f2, W_pos1, b_pos1, mlps1, gamma_bn1, beta_bn1)
    feat1_new = _conv1d(feat1_new, W_t1, b_t1)
    f2b = _conv1d_pallas(feat2, W_t11, b_t11)
    f1b = _conv1d_pallas(feat1, W_t22, b_t22)
    feat2_new = _cross(pc2, pc1, f2b, f1b, W_pos1, b_pos1, mlps1, gamma_bn1, beta_bn1)
    feat2_new = _conv1d(feat2_new, W_t2, b_t2)
    feat1_final = _cross(pc1, pc2, feat1_new, feat2_new, W_pos2, b_pos2, mlps2, gamma_bn2, beta_bn2)
    return (feat1_new, feat2_new, feat1_final)


# speedup vs baseline: 10.0048x; 10.0048x over previous
import functools

import jax
import jax.numpy as jnp
from jax import lax
from jax.experimental import pallas as pl
from jax.experimental.pallas import tpu as pltpu
from jax.experimental.pallas import tpu_sc as plsc

Bv, Nv, Cv = 2, 2048, 64
NS = 16
G = 4
f32 = jnp.float32
i32 = jnp.int32

NW = 32
ROWS = Bv * Nv
RPW = ROWS // NW
CH = 16
NCH = RPW // CH


def _tc_transforms(feat1T, feat2T, Wt11, b11, Wt22, b22):
    N, C = Nv, Cv

    def body(f1_ref, f2_ref, w11_ref, b11_ref, w22_ref, b22_ref,
             t11_ref, t22_ref):
        f1 = f1_ref[...]
        f2 = f2_ref[...]
        for w_ref, b_ref, o_ref in ((w11_ref, b11_ref, t11_ref),
                                    (w22_ref, b22_ref, t22_ref)):
            w = w_ref[...]
            b = b_ref[...][None, :]
            o_ref[pl.ds(0, N), :] = jnp.dot(f1, w, preferred_element_type=f32) + b
            o_ref[pl.ds(N, N), :] = jnp.dot(f2, w, preferred_element_type=f32) + b

    out = pl.pallas_call(
        body,
        grid=(Bv,),
        in_specs=[
            pl.BlockSpec((None, N, C), lambda b: (b, 0, 0)),
            pl.BlockSpec((None, N, C), lambda b: (b, 0, 0)),
            pl.BlockSpec((C, C), lambda b: (0, 0)),
            pl.BlockSpec((C,), lambda b: (0,)),
            pl.BlockSpec((C, C), lambda b: (0, 0)),
            pl.BlockSpec((C,), lambda b: (0,)),
        ],
        out_specs=[
            pl.BlockSpec((None, 2 * N, C), lambda b: (b, 0, 0)),
            pl.BlockSpec((None, 2 * N, C), lambda b: (b, 0, 0)),
        ],
        out_shape=[
            jax.ShapeDtypeStruct((Bv, 2 * N, C), f32),
            jax.ShapeDtypeStruct((Bv, 2 * N, C), f32),
        ],
    )(feat1T, feat2T, Wt11, b11, Wt22, b22)
    return out


def _tc_dists(pc1t, pc2p, pc2t, pc1p, pc2ps, pc1ps):
    N = Nv
    NB = 512

    def body(a1_ref, b2_ref, a2_ref, b1_ref, b2s_ref, b1s_ref,
             g12_ref, g21_ref, g12s_ref, g21s_ref):
        for b_ref, a_ref, o_ref in ((b2_ref, a1_ref, g12_ref),
                                    (b1_ref, a2_ref, g21_ref),
                                    (b2s_ref, a1_ref, g12s_ref),
                                    (b1s_ref, a2_ref, g21s_ref)):
            p2 = b_ref[...]
            n2 = jnp.sum(p2 * p2, axis=0, keepdims=True)
            o_ref[...] = n2 - 2.0 * jnp.dot(
                a_ref[...], p2, precision=lax.Precision.HIGHEST,
                preferred_element_type=f32)

    blk_a = pl.BlockSpec((None, NB, 16), lambda b, i: (b, i, 0))
    blk_p = pl.BlockSpec((None, 16, N), lambda b, i: (b, 0, 0))
    blk_o = pl.BlockSpec((None, NB, N), lambda b, i: (b, i, 0))
    return pl.pallas_call(
        body,
        grid=(Bv, N // NB),
        in_specs=[blk_a, blk_p, blk_a, blk_p, blk_p, blk_p],
        out_specs=[blk_o, blk_o, blk_o, blk_o],
        out_shape=[jax.ShapeDtypeStruct((Bv, N, N), f32)] * 4,
    )(pc1t, pc2p, pc2t, pc1p, pc2ps, pc1ps)


def _bf_min(v, iota):
    for d in (8, 4, 2, 1):
        v = jnp.minimum(v, v[iota ^ d])
    return v


def _sc_topk_gather(dist, dist_s, tab, xyzt):
    mesh = plsc.VectorSubcoreMesh(core_axis_name="c", subcore_axis_name="s",
                                  num_cores=2, num_subcores=16)
    CH8 = 8
    NCH8 = RPW // CH8

    @functools.partial(
        pl.kernel,
        out_type=[
            jax.ShapeDtypeStruct((ROWS * NS // 128, 128), i32),
            jax.ShapeDtypeStruct((ROWS * NS, Cv), f32),
            jax.ShapeDtypeStruct((ROWS * NS, 16), f32),
        ],
        mesh=mesh,
        scratch_types=[
            pltpu.VMEM((2, CH8, Nv), f32),
            pltpu.VMEM((2, CH8, Nv), f32),
            pltpu.VMEM((1, 128), i32),
            pltpu.VMEM((CH8 * NS, Cv), f32),
            pltpu.VMEM((CH8 * NS, 16), f32),
            pltpu.SemaphoreType.DMA,
            pltpu.SemaphoreType.DMA,
            pltpu.SemaphoreType.DMA,
        ],
        compiler_params=pltpu.CompilerParams(use_tc_tiling_on_sc=False),
        name="sc_topk_gather",
    )
    def k(dist_h, dists_h, tab_h, xyzt_h, idx_o, feat_o, xyz_o,
          dbuf, sbuf, ibuf, fbuf, xbuf, dsem_a, dsem_b, gsem):
        cid = lax.axis_index("c")
        sid = lax.axis_index("s")
        wid = sid * 2 + cid
        row0 = wid * RPW
        boff = (wid // 16) * Nv
        iota = lax.iota(i32, 16)
        inf16 = lax.broadcast(jnp.float32(jnp.inf), (16,))

        def fetch(ci, slot, sem):
            h1 = pltpu.async_copy(
                dist_h.at[pl.ds(row0 + ci * CH8, CH8)], dbuf.at[slot], sem)
            h2 = pltpu.async_copy(
                dists_h.at[pl.ds(row0 + ci * CH8, CH8)], sbuf.at[slot], sem)
            return (h1, h2)

        dsems = (dsem_a, dsem_b)
        handles = [None, None]
        handles[0] = fetch(0, 0, dsems[0])

        def topk_row(r, slot):
            ms = []
            for q in range(8):
                m = sbuf[slot, r, pl.ds(q * 256, 16)]
                for i in range(1, 16):
                    m = jnp.minimum(m, sbuf[slot, r, pl.ds(q * 256 + i * 16, 16)])
                ms.append(m)

            def ext(it, carry):
                a_v = carry[0]
                ms = list(carry[1:])
                m = ms[0]
                p = iota * 0
                for q in range(1, 8):
                    t = ms[q] < m
                    m = jnp.where(t, ms[q], m)
                    p = jnp.where(t, q, p)
                vstar = _bf_min(m, iota)
                lstar = _bf_min(jnp.where(m == vstar, iota, 16), iota)
                qstar = p[lstar]
                cvec = qstar * 16 + lstar
                cstar = cvec[0]
                V = dbuf[slot, r, pl.ds(cstar * 16, 16)]
                istar = _bf_min(jnp.where(V == vstar, iota, 16), iota)
                a_v = jnp.where(iota == it, cvec * 16 + istar, a_v)
                Vm = jnp.where(iota == istar, inf16, V)
                dbuf[slot, r, pl.ds(cstar * 16, 16)] = Vm
                newmin = _bf_min(Vm, iota)
                cond_l = iota == lstar
                for q in range(8):
                    ms[q] = jnp.where((qstar == q) & cond_l, newmin, ms[q])
                return tuple([a_v] + ms)

            out = lax.fori_loop(0, 16, ext, tuple([iota * 0] + ms))
            ibuf[0, pl.ds(r * 16, 16)] = out[0] + boff
            return slot

        for ci in range(NCH8):
            slot = ci % 2
            if ci + 1 < NCH8:
                handles[(ci + 1) % 2] = fetch(ci + 1, (ci + 1) % 2,
                                              dsems[(ci + 1) % 2])
            handles[slot][0].wait()
            handles[slot][1].wait()
            lax.fori_loop(0, CH8, topk_row, slot)
            h0 = pltpu.async_copy(tab_h.at[ibuf.at[0]], fbuf, gsem)
            h2 = pltpu.async_copy(xyzt_h.at[ibuf.at[0]], xbuf, gsem)
            h0.wait()
            h2.wait()
            out0 = (row0 + ci * CH8) * NS
            pltpu.sync_copy(fbuf, feat_o.at[pl.ds(out0, CH8 * NS)])
            pltpu.sync_copy(xbuf, xyz_o.at[pl.ds(out0, CH8 * NS)])
            pltpu.sync_copy(ibuf, idx_o.at[pl.ds(wid * NCH8 + ci, 1)])

    return k(dist, dist_s, tab, xyzt)


def _sc_gather(idx, tab):
    mesh = plsc.VectorSubcoreMesh(core_axis_name="c", subcore_axis_name="s",
                                  num_cores=2, num_subcores=16)
    NR = idx.shape[0] // NW

    @functools.partial(
        pl.kernel,
        out_type=jax.ShapeDtypeStruct((ROWS * NS, Cv), f32),
        mesh=mesh,
        scratch_types=[
            pltpu.VMEM((1, 128), i32),
            pltpu.VMEM((128, Cv), f32),
            pltpu.SemaphoreType.DMA,
        ],
        compiler_params=pltpu.CompilerParams(use_tc_tiling_on_sc=False),
        name="sc_gather_only",
    )
    def k2(idx_h, tab_h, feat_o, ibuf, fbuf, gsem):
        cid = lax.axis_index("c")
        sid = lax.axis_index("s")
        wid = sid * 2 + cid

        def body(ci, _):
            rrow = wid * NR + ci
            pltpu.sync_copy(idx_h.at[pl.ds(rrow, 1)], ibuf)
            pltpu.async_copy(tab_h.at[ibuf.at[0]], fbuf, gsem).wait()
            pltpu.sync_copy(fbuf, feat_o.at[pl.ds(rrow * 128, 128)])
            return 0

        lax.fori_loop(0, NR, body, 0)

    return k2(idx, tab)


def _tc_dense(gf_flat, gx_flat, p1t, pc1t, W_pos, b_pos, gamma0, beta0,
              mlps, final):
    N, C = Nv, Cv
    M4 = N * NS // 4
    C4 = 4 * C

    gf = gf_flat.reshape(Bv, M4, C4)
    gx = gx_flat.reshape(Bv, M4, 64)
    p1t4 = jnp.tile(p1t, (1, 1, 4))
    pc1t4 = jnp.tile(pc1t, (1, 1, 4))
    eye4 = jnp.eye(4, dtype=f32)
    posW4 = jnp.kron(eye4, jnp.pad(W_pos.T, ((0, 13), (0, 0))))
    posb4 = jnp.tile(b_pos, 4)
    g0_4 = jnp.tile(gamma0, 4)
    b0_4 = jnp.tile(beta0, 4)
    ch = (jnp.arange(C4) % C) // (C // G)
    mmat = (ch[:, None] == ch[None, :]).astype(f32)
    mlp_packed = []
    for (W, b, gg, bb) in mlps:
        mlp_packed.append((jnp.kron(eye4, W.T), jnp.tile(b, 4),
                           jnp.tile(gg, 4), jnp.tile(bb, 4)))
    n_mlp = len(mlps)
    cnt = float(M4 * 64)

    def gn_leaky(x, a_g, a_b, mm):
        s1 = jnp.sum(x, axis=0, keepdims=True)
        s2 = jnp.sum(x * x, axis=0, keepdims=True)
        mean = jnp.dot(s1, mm, preferred_element_type=f32) / cnt
        ex2 = jnp.dot(s2, mm, preferred_element_type=f32) / cnt
        var = ex2 - mean * mean
        a = a_g[None, :] * lax.rsqrt(var + 1e-5)
        d = a_b[None, :] - mean * a
        y = x * a + d
        return jnp.maximum(y, 0.1 * y)

    def body(*refs):
        (gf_ref, gx_ref, p1t4_ref, pc1t4_ref, posw_ref, posb_ref,
         g0_ref, b0_ref, mm_ref) = refs[:9]
        pos = 9
        mlp_refs = refs[pos:pos + 4 * n_mlp]
        pos += 4 * n_mlp
        if final is not None:
            wf_ref, bf_ref = refs[pos:pos + 2]
            pos += 2
        out_ref, outt_ref = refs[pos:pos + 2]

        mm = mm_ref[...]
        dx3 = gx_ref[...].reshape(N, 4, 64) - pc1t4_ref[...][:, None, :]
        p = jnp.dot(dx3.reshape(M4, 64), posw_ref[...],
                    preferred_element_type=f32) + posb_ref[...][None, :]
        x3 = (gf_ref[...] + p).reshape(N, 4, C4) + p1t4_ref[...][:, None, :]
        h = gn_leaky(x3.reshape(M4, C4), g0_ref[...], b0_ref[...], mm)
        for li in range(n_mlp):
            w4_ref, b4_ref, gg_ref, bb_ref = mlp_refs[4 * li:4 * li + 4]
            hc = jnp.dot(h, w4_ref[...],
                         preferred_element_type=f32) + b4_ref[...][None, :]
            h = gn_leaky(hc, gg_ref[...], bb_ref[...], mm)
        m = jnp.maximum(jnp.maximum(h[:, 0:C], h[:, C:2 * C]),
                        jnp.maximum(h[:, 2 * C:3 * C], h[:, 3 * C:4 * C]))
        hm = jnp.max(m.reshape(N, 4, C), axis=1)
        if final is not None:
            ot = jnp.dot(hm, wf_ref[...],
                         preferred_element_type=f32) + bf_ref[...][None, :]
        else:
            ot = hm
        outt_ref[...] = ot
        out_ref[...] = ot.T

    inputs = [gf, gx, p1t4, pc1t4, posW4, posb4, g0_4, b0_4, mmat]
    in_specs = [
        pl.BlockSpec((None, M4, C4), lambda b: (b, 0, 0)),
        pl.BlockSpec((None, M4, 64), lambda b: (b, 0, 0)),
        pl.BlockSpec((None, N, C4), lambda b: (b, 0, 0)),
        pl.BlockSpec((None, N, 64), lambda b: (b, 0, 0)),
        pl.BlockSpec((64, C4), lambda b: (0, 0)),
        pl.BlockSpec((C4,), lambda b: (0,)),
        pl.BlockSpec((C4,), lambda b: (0,)),
        pl.BlockSpec((C4,), lambda b: (0,)),
        pl.BlockSpec((C4, C4), lambda b: (0, 0)),
    ]
    for tup in mlp_packed:
        inputs.extend(tup)
        in_specs.extend([
            pl.BlockSpec((C4, C4), lambda b: (0, 0)),
            pl.BlockSpec((C4,), lambda b: (0,)),
            pl.BlockSpec((C4,), lambda b: (0,)),
            pl.BlockSpec((C4,), lambda b: (0,)),
        ])
    if final is not None:
        W_f, b_f = final
        inputs.extend([W_f.T, b_f])
        in_specs.extend([
            pl.BlockSpec((C, C), lambda b: (0, 0)),
            pl.BlockSpec((C,), lambda b: (0,)),
        ])

    out, outt = pl.pallas_call(
        body,
        grid=(Bv,),
        in_specs=in_specs,
        out_specs=[
            pl.BlockSpec((None, C, N), lambda b: (b, 0, 0)),
            pl.BlockSpec((None, N, C), lambda b: (b, 0, 0)),
        ],
        out_shape=[
            jax.ShapeDtypeStruct((Bv, C, N), f32),
            jax.ShapeDtypeStruct((Bv, N, C), f32),
        ],
    )(*inputs)
    return out, outt


def kernel(pc1, pc2, feat1, feat2, W_t11, b_t11, W_t22, b_t22, W_pos1, b_pos1, gamma_bn1, beta_bn1, Wm1_1, bm1_1, gm1_1, btm1_1, Wm1_2, bm1_2, gm1_2, btm1_2, W_t1, b_t1, W_t2, b_t2, W_pos2, b_pos2, gamma_bn2, beta_bn2, Wm2_1, bm2_1, gm2_1, btm2_1):
    N, C = Nv, Cv

    feat1T = jnp.transpose(feat1, (0, 2, 1))
    feat2T = jnp.transpose(feat2, (0, 2, 1))
    pc1t = jnp.pad(jnp.transpose(pc1, (0, 2, 1)), ((0, 0), (0, 0), (0, 13)))
    pc2t = jnp.pad(jnp.transpose(pc2, (0, 2, 1)), ((0, 0), (0, 0), (0, 13)))
    pc1p = jnp.pad(pc1, ((0, 0), (0, 13), (0, 0)))
    pc2p = jnp.pad(pc2, ((0, 0), (0, 13), (0, 0)))

    pp = jnp.arange(N)
    perm = (pp // 256) * 256 + (pp % 16) * 16 + (pp % 256) // 16
    pc2ps = pc2p[:, :, perm]
    pc1ps = pc1p[:, :, perm]

    T11, T22 = _tc_transforms(feat1T, feat2T, W_t11.T, b_t11, W_t22.T, b_t22)
    G12, G21, G12s, G21s = _tc_dists(pc1t, pc2p, pc2t, pc1p, pc2ps, pc1ps)

    f2t = T22[:, N:, :].reshape(Bv * N, C)
    f1bt = T22[:, :N, :].reshape(Bv * N, C)
    p1t_1 = T11[:, :N, :]
    p1t_2 = T11[:, N:, :]

    idx12, gf1, gx1 = _sc_topk_gather(G12.reshape(Bv * N, N),
                                      G12s.reshape(Bv * N, N), f2t,
                                      pc2t.reshape(Bv * N, 16))
    idx21, gf2, gx2 = _sc_topk_gather(G21.reshape(Bv * N, N),
                                      G21s.reshape(Bv * N, N), f1bt,
                                      pc1t.reshape(Bv * N, 16))

    mlp1 = [(Wm1_1, bm1_1, gm1_1, btm1_1), (Wm1_2, bm1_2, gm1_2, btm1_2)]
    feat1_new, feat1_new_t = _tc_dense(
        gf1, gx1, p1t_1, pc1t, W_pos1, b_pos1, gamma_bn1, beta_bn1, mlp1,
        (W_t1, b_t1))
    feat2_new, feat2_new_t = _tc_dense(
        gf2, gx2, p1t_2, pc2t, W_pos1, b_pos1, gamma_bn1, beta_bn1, mlp1,
        (W_t2, b_t2))

    gf3 = _sc_gather(idx12, feat2_new_t.reshape(Bv * N, C))
    mlp2 = [(Wm2_1, bm2_1, gm2_1, btm2_1)]
    feat1_final, _ = _tc_dense(
        gf3, gx1, feat1_new_t, pc1t, W_pos2, b_pos2, gamma_bn2, beta_bn2,
        mlp2, None)

    return (feat1_new, feat2_new, feat1_final)


# speedup vs baseline: 10.0558x; 1.0051x over previous
import functools

import jax
import jax.numpy as jnp
from jax import lax
from jax.experimental import pallas as pl
from jax.experimental.pallas import tpu as pltpu
from jax.experimental.pallas import tpu_sc as plsc

Bv, Nv, Cv = 2, 2048, 64
NS = 16
G = 4
f32 = jnp.float32
i32 = jnp.int32

NW = 32
ROWS = Bv * Nv
RPW = ROWS // NW
CH = 16
NCH = RPW // CH


def _tc_transforms(feat1, feat2, Wt11, b11, Wt22, b22):
    N, C = Nv, Cv

    def body(f1_ref, f2_ref, w11_ref, b11_ref, w22_ref, b22_ref,
             f1t_ref, f2bt_ref, f1bt_ref, f2t_ref):
        dn = (((0,), (0,)), ((), ()))
        for w_ref, b_ref, o1_ref, o2_ref in (
                (w11_ref, b11_ref, f1t_ref, f2bt_ref),
                (w22_ref, b22_ref, f1bt_ref, f2t_ref)):
            w = w_ref[...]
            b = b_ref[...][None, :]
            o1_ref[...] = lax.dot_general(
                f1_ref[...], w, dn, preferred_element_type=f32) + b
            o2_ref[...] = lax.dot_general(
                f2_ref[...], w, dn, preferred_element_type=f32) + b

    blk_f = pl.BlockSpec((None, C, N), lambda b: (b, 0, 0))
    blk_w = pl.BlockSpec((C, C), lambda b: (0, 0))
    blk_b = pl.BlockSpec((C,), lambda b: (0,))
    blk_o = pl.BlockSpec((None, N, C), lambda b: (b, 0, 0))
    return pl.pallas_call(
        body,
        grid=(Bv,),
        in_specs=[blk_f, blk_f, blk_w, blk_b, blk_w, blk_b],
        out_specs=[blk_o, blk_o, blk_o, blk_o],
        out_shape=[jax.ShapeDtypeStruct((Bv, N, C), f32)] * 4,
    )(feat1, feat2, Wt11, b11, Wt22, b22)


def _tc_dists(pc1t, pc2p, pc2t, pc1p, pc2ps, pc1ps):
    N = Nv
    NB = 512

    def body(a1_ref, b2_ref, a2_ref, b1_ref, b2s_ref, b1s_ref,
             g12_ref, g21_ref, g12s_ref, g21s_ref):
        for b_ref, a_ref, o_ref in ((b2_ref, a1_ref, g12_ref),
                                    (b1_ref, a2_ref, g21_ref),
                                    (b2s_ref, a1_ref, g12s_ref),
                                    (b1s_ref, a2_ref, g21s_ref)):
            p2 = b_ref[...]
            n2 = jnp.sum(p2 * p2, axis=0, keepdims=True)
            o_ref[...] = n2 - 2.0 * jnp.dot(
                a_ref[...], p2, precision=lax.Precision.HIGHEST,
                preferred_element_type=f32)

    blk_a = pl.BlockSpec((None, NB, 16), lambda b, i: (b, i, 0))
    blk_p = pl.BlockSpec((None, 16, N), lambda b, i: (b, 0, 0))
    blk_o = pl.BlockSpec((None, NB, N), lambda b, i: (b, i, 0))
    return pl.pallas_call(
        body,
        grid=(Bv, N // NB),
        in_specs=[blk_a, blk_p, blk_a, blk_p, blk_p, blk_p],
        out_specs=[blk_o, blk_o, blk_o, blk_o],
        out_shape=[jax.ShapeDtypeStruct((Bv, N, N), f32)] * 4,
    )(pc1t, pc2p, pc2t, pc1p, pc2ps, pc1ps)


def _bf_min(v, iota):
    for d in (8, 4, 2, 1):
        v = jnp.minimum(v, v[iota ^ d])
    return v


def _sc_topk_gather(dist, dist_s, tab, xyzt):
    mesh = plsc.VectorSubcoreMesh(core_axis_name="c", subcore_axis_name="s",
                                  num_cores=2, num_subcores=16)
    CH8 = 8
    NCH8 = RPW // CH8

    @functools.partial(
        pl.kernel,
        out_type=[
            jax.ShapeDtypeStruct((ROWS * NS // 128, 128), i32),
            jax.ShapeDtypeStruct((ROWS * NS, Cv), f32),
            jax.ShapeDtypeStruct((ROWS * NS, 16), f32),
        ],
        mesh=mesh,
        scratch_types=[
            pltpu.VMEM((2, CH8, Nv), f32),
            pltpu.VMEM((2, CH8, Nv), f32),
            pltpu.VMEM((1, 128), i32),
            pltpu.VMEM((CH8 * NS, Cv), f32),
            pltpu.VMEM((CH8 * NS, 16), f32),
            pltpu.SemaphoreType.DMA,
            pltpu.SemaphoreType.DMA,
            pltpu.SemaphoreType.DMA,
        ],
        compiler_params=pltpu.CompilerParams(use_tc_tiling_on_sc=False),
        name="sc_topk_gather",
    )
    def k(dist_h, dists_h, tab_h, xyzt_h, idx_o, feat_o, xyz_o,
          dbuf, sbuf, ibuf, fbuf, xbuf, dsem_a, dsem_b, gsem):
        cid = lax.axis_index("c")
        sid = lax.axis_index("s")
        wid = sid * 2 + cid
        row0 = wid * RPW
        boff = (wid // 16) * Nv
        iota = lax.iota(i32, 16)
        inf16 = lax.broadcast(jnp.float32(jnp.inf), (16,))

        def fetch(ci, slot, sem):
            h1 = pltpu.async_copy(
                dist_h.at[pl.ds(row0 + ci * CH8, CH8)], dbuf.at[slot], sem)
            h2 = pltpu.async_copy(
                dists_h.at[pl.ds(row0 + ci * CH8, CH8)], sbuf.at[slot], sem)
            return (h1, h2)

        dsems = (dsem_a, dsem_b)
        handles = [None, None]
        handles[0] = fetch(0, 0, dsems[0])

        def topk_row(r, slot):
            ms = []
            for q in range(8):
                m = sbuf[slot, r, pl.ds(q * 256, 16)]
                for i in range(1, 16):
                    m = jnp.minimum(m, sbuf[slot, r, pl.ds(q * 256 + i * 16, 16)])
                ms.append(m)

            def ext(it, carry):
                a_v = carry[0]
                ms = list(carry[1:])
                m = ms[0]
                p = iota * 0
                for q in range(1, 8):
                    t = ms[q] < m
                    m = jnp.where(t, ms[q], m)
                    p = jnp.where(t, q, p)
                vstar = _bf_min(m, iota)
                lstar = _bf_min(jnp.where(m == vstar, iota, 16), iota)
                qstar = p[lstar]
                cvec = qstar * 16 + lstar
                cstar = cvec[0]
                V = dbuf[slot, r, pl.ds(cstar * 16, 16)]
                istar = _bf_min(jnp.where(V == vstar, iota, 16), iota)
                a_v = jnp.where(iota == it, cvec * 16 + istar, a_v)
                Vm = jnp.where(iota == istar, inf16, V)
                dbuf[slot, r, pl.ds(cstar * 16, 16)] = Vm
                newmin = _bf_min(Vm, iota)
                cond_l = iota == lstar
                for q in range(8):
                    ms[q] = jnp.where((qstar == q) & cond_l, newmin, ms[q])
                return tuple([a_v] + ms)

            out = lax.fori_loop(0, 16, ext, tuple([iota * 0] + ms))
            ibuf[0, pl.ds(r * 16, 16)] = out[0] + boff
            return slot

        for ci in range(NCH8):
            slot = ci % 2
            if ci + 1 < NCH8:
                handles[(ci + 1) % 2] = fetch(ci + 1, (ci + 1) % 2,
                                              dsems[(ci + 1) % 2])
            handles[slot][0].wait()
            handles[slot][1].wait()
            lax.fori_loop(0, CH8, topk_row, slot)
            h0 = pltpu.async_copy(tab_h.at[ibuf.at[0]], fbuf, gsem)
            h2 = pltpu.async_copy(xyzt_h.at[ibuf.at[0]], xbuf, gsem)
            h0.wait()
            h2.wait()
            out0 = (row0 + ci * CH8) * NS
            pltpu.sync_copy(fbuf, feat_o.at[pl.ds(out0, CH8 * NS)])
            pltpu.sync_copy(xbuf, xyz_o.at[pl.ds(out0, CH8 * NS)])
            pltpu.sync_copy(ibuf, idx_o.at[pl.ds(wid * NCH8 + ci, 1)])

    return k(dist, dist_s, tab, xyzt)


def _sc_gather(idx, tab):
    mesh = plsc.VectorSubcoreMesh(core_axis_name="c", subcore_axis_name="s",
                                  num_cores=2, num_subcores=16)
    NR = idx.shape[0] // NW

    @functools.partial(
        pl.kernel,
        out_type=jax.ShapeDtypeStruct((ROWS * NS, Cv), f32),
        mesh=mesh,
        scratch_types=[
            pltpu.VMEM((1, 128), i32),
            pltpu.VMEM((128, Cv), f32),
            pltpu.SemaphoreType.DMA,
        ],
        compiler_params=pltpu.CompilerParams(use_tc_tiling_on_sc=False),
        name="sc_gather_only",
    )
    def k2(idx_h, tab_h, feat_o, ibuf, fbuf, gsem):
        cid = lax.axis_index("c")
        sid = lax.axis_index("s")
        wid = sid * 2 + cid

        def body(ci, _):
            rrow = wid * NR + ci
            pltpu.sync_copy(idx_h.at[pl.ds(rrow, 1)], ibuf)
            pltpu.async_copy(tab_h.at[ibuf.at[0]], fbuf, gsem).wait()
            pltpu.sync_copy(fbuf, feat_o.at[pl.ds(rrow * 128, 128)])
            return 0

        lax.fori_loop(0, NR, body, 0)

    return k2(idx, tab)


def _tc_dense(gf_flat, gx_flat, p1t, pc1t, W_pos, b_pos, gamma0, beta0,
              mlps, final):
    N, C = Nv, Cv
    M4 = N * NS // 4
    C4 = 4 * C

    gf = gf_flat.reshape(Bv, M4, C4)
    gx = gx_flat.reshape(Bv, M4, 64)
    p1t4 = jnp.tile(p1t, (1, 1, 4))
    pc1t4 = jnp.tile(pc1t, (1, 1, 4))
    eye4 = jnp.eye(4, dtype=f32)
    posW4 = jnp.kron(eye4, jnp.pad(W_pos.T, ((0, 13), (0, 0))))
    posb4 = jnp.tile(b_pos, 4)
    g0_4 = jnp.tile(gamma0, 4)
    b0_4 = jnp.tile(beta0, 4)
    ch = (jnp.arange(C4) % C) // (C // G)
    mmat = (ch[:, None] == ch[None, :]).astype(f32)
    mlp_packed = []
    for (W, b, gg, bb) in mlps:
        mlp_packed.append((jnp.kron(eye4, W.T), jnp.tile(b, 4),
                           jnp.tile(gg, 4), jnp.tile(bb, 4)))
    n_mlp = len(mlps)
    cnt = float(M4 * 64)

    def gn_leaky(x, a_g, a_b, mm):
        s1 = jnp.sum(x, axis=0, keepdims=True)
        s2 = jnp.sum(x * x, axis=0, keepdims=True)
        mean = jnp.dot(s1, mm, preferred_element_type=f32) / cnt
        ex2 = jnp.dot(s2, mm, preferred_element_type=f32) / cnt
        var = ex2 - mean * mean
        a = a_g[None, :] * lax.rsqrt(var + 1e-5)
        d = a_b[None, :] - mean * a
        y = x * a + d
        return jnp.maximum(y, 0.1 * y)

    def body(*refs):
        (gf_ref, gx_ref, p1t4_ref, pc1t4_ref, posw_ref, posb_ref,
         g0_ref, b0_ref, mm_ref) = refs[:9]
        pos = 9
        mlp_refs = refs[pos:pos + 4 * n_mlp]
        pos += 4 * n_mlp
        if final is not None:
            wf_ref, bf_ref = refs[pos:pos + 2]
            pos += 2
        out_ref, outt_ref = refs[pos:pos + 2]

        mm = mm_ref[...]
        dx3 = gx_ref[...].reshape(N, 4, 64) - pc1t4_ref[...][:, None, :]
        p = jnp.dot(dx3.reshape(M4, 64), posw_ref[...],
                    preferred_element_type=f32) + posb_ref[...][None, :]
        x3 = (gf_ref[...] + p).reshape(N, 4, C4) + p1t4_ref[...][:, None, :]
        h = gn_leaky(x3.reshape(M4, C4), g0_ref[...], b0_ref[...], mm)
        for li in range(n_mlp):
            w4_ref, b4_ref, gg_ref, bb_ref = mlp_refs[4 * li:4 * li + 4]
            hc = jnp.dot(h, w4_ref[...],
                         preferred_element_type=f32) + b4_ref[...][None, :]
            h = gn_leaky(hc, gg_ref[...], bb_ref[...], mm)
        m = jnp.maximum(jnp.maximum(h[:, 0:C], h[:, C:2 * C]),
                        jnp.maximum(h[:, 2 * C:3 * C], h[:, 3 * C:4 * C]))
        hm = jnp.max(m.reshape(N, 4, C), axis=1)
        if final is not None:
            ot = jnp.dot(hm, wf_ref[...],
                         preferred_element_type=f32) + bf_ref[...][None, :]
        else:
            ot = hm
        outt_ref[...] = ot
        out_ref[...] = ot.T

    inputs = [gf, gx, p1t4, pc1t4, posW4, posb4, g0_4, b0_4, mmat]
    in_specs = [
        pl.BlockSpec((None, M4, C4), lambda b: (b, 0, 0)),
        pl.BlockSpec((None, M4, 64), lambda b: (b, 0, 0)),
        pl.BlockSpec((None, N, C4), lambda b: (b, 0, 0)),
        pl.BlockSpec((None, N, 64), lambda b: (b, 0, 0)),
        pl.BlockSpec((64, C4), lambda b: (0, 0)),
        pl.BlockSpec((C4,), lambda b: (0,)),
        pl.BlockSpec((C4,), lambda b: (0,)),
        pl.BlockSpec((C4,), lambda b: (0,)),
        pl.BlockSpec((C4, C4), lambda b: (0, 0)),
    ]
    for tup in mlp_packed:
        inputs.extend(tup)
        in_specs.extend([
            pl.BlockSpec((C4, C4), lambda b: (0, 0)),
            pl.BlockSpec((C4,), lambda b: (0,)),
            pl.BlockSpec((C4,), lambda b: (0,)),
            pl.BlockSpec((C4,), lambda b: (0,)),
        ])
    if final is not None:
        W_f, b_f = final
        inputs.extend([W_f.T, b_f])
        in_specs.extend([
            pl.BlockSpec((C, C), lambda b: (0, 0)),
            pl.BlockSpec((C,), lambda b: (0,)),
        ])

    out, outt = pl.pallas_call(
        body,
        grid=(Bv,),
        in_specs=in_specs,
        out_specs=[
            pl.BlockSpec((None, C, N), lambda b: (b, 0, 0)),
            pl.BlockSpec((None, N, C), lambda b: (b, 0, 0)),
        ],
        out_shape=[
            jax.ShapeDtypeStruct((Bv, C, N), f32),
            jax.ShapeDtypeStruct((Bv, N, C), f32),
        ],
    )(*inputs)
    return out, outt


def kernel(pc1, pc2, feat1, feat2, W_t11, b_t11, W_t22, b_t22, W_pos1, b_pos1, gamma_bn1, beta_bn1, Wm1_1, bm1_1, gm1_1, btm1_1, Wm1_2, bm1_2, gm1_2, btm1_2, W_t1, b_t1, W_t2, b_t2, W_pos2, b_pos2, gamma_bn2, beta_bn2, Wm2_1, bm2_1, gm2_1, btm2_1):
    N, C = Nv, Cv

    pc1t = jnp.pad(jnp.transpose(pc1, (0, 2, 1)), ((0, 0), (0, 0), (0, 13)))
    pc2t = jnp.pad(jnp.transpose(pc2, (0, 2, 1)), ((0, 0), (0, 0), (0, 13)))
    pc1p = jnp.pad(pc1, ((0, 0), (0, 13), (0, 0)))
    pc2p = jnp.pad(pc2, ((0, 0), (0, 13), (0, 0)))

    pp = jnp.arange(N)
    perm = (pp // 256) * 256 + (pp % 16) * 16 + (pp % 256) // 16
    pc2ps = pc2p[:, :, perm]
    pc1ps = pc1p[:, :, perm]

    p1t_1, p1t_2, f1bt3, f2t3 = _tc_transforms(
        feat1, feat2, W_t11.T, b_t11, W_t22.T, b_t22)
    G12, G21, G12s, G21s = _tc_dists(pc1t, pc2p, pc2t, pc1p, pc2ps, pc1ps)

    f2t = f2t3.reshape(Bv * N, C)
    f1bt = f1bt3.reshape(Bv * N, C)

    idx12, gf1, gx1 = _sc_topk_gather(G12.reshape(Bv * N, N),
                                      G12s.reshape(Bv * N, N), f2t,
                                      pc2t.reshape(Bv * N, 16))
    idx21, gf2, gx2 = _sc_topk_gather(G21.reshape(Bv * N, N),
                                      G21s.reshape(Bv * N, N), f1bt,
                                      pc1t.reshape(Bv * N, 16))

    mlp1 = [(Wm1_1, bm1_1, gm1_1, btm1_1), (Wm1_2, bm1_2, gm1_2, btm1_2)]
    feat1_new, feat1_new_t = _tc_dense(
        gf1, gx1, p1t_1, pc1t, W_pos1, b_pos1, gamma_bn1, beta_bn1, mlp1,
        (W_t1, b_t1))
    feat2_new, feat2_new_t = _tc_dense(
        gf2, gx2, p1t_2, pc2t, W_pos1, b_pos1, gamma_bn1, beta_bn1, mlp1,
        (W_t2, b_t2))

    gf3 = _sc_gather(idx12, feat2_new_t.reshape(Bv * N, C))
    mlp2 = [(Wm2_1, bm2_1, gm2_1, btm2_1)]
    feat1_final, _ = _tc_dense(
        gf3, gx1, feat1_new_t, pc1t, W_pos2, b_pos2, gamma_bn2, beta_bn2,
        mlp2, None)

    return (feat1_new, feat2_new, feat1_final)


# speedup vs baseline: 11.9580x; 1.1892x over previous
import functools

import jax
import jax.numpy as jnp
from jax import lax
from jax.experimental import pallas as pl
from jax.experimental.pallas import tpu as pltpu
from jax.experimental.pallas import tpu_sc as plsc

Bv, Nv, Cv = 2, 2048, 64
NS = 16
G = 4
f32 = jnp.float32
i32 = jnp.int32

NW = 32
ROWS = Bv * Nv
RPW = ROWS // NW
CH = 16
NCH = RPW // CH


def _tc_transforms(feat1, feat2, Wt11, b11, Wt22, b22):
    N, C = Nv, Cv

    def body(f1_ref, f2_ref, w11_ref, b11_ref, w22_ref, b22_ref,
             f1t_ref, f2bt_ref, f1bt_ref, f2t_ref):
        dn = (((0,), (0,)), ((), ()))
        for w_ref, b_ref, o1_ref, o2_ref in (
                (w11_ref, b11_ref, f1t_ref, f2bt_ref),
                (w22_ref, b22_ref, f1bt_ref, f2t_ref)):
            w = w_ref[...]
            b = b_ref[...][None, :]
            o1_ref[...] = lax.dot_general(
                f1_ref[...], w, dn, preferred_element_type=f32) + b
            o2_ref[...] = lax.dot_general(
                f2_ref[...], w, dn, preferred_element_type=f32) + b

    blk_f = pl.BlockSpec((None, C, N), lambda b: (b, 0, 0))
    blk_w = pl.BlockSpec((C, C), lambda b: (0, 0))
    blk_b = pl.BlockSpec((C,), lambda b: (0,))
    blk_o = pl.BlockSpec((None, N, C), lambda b: (b, 0, 0))
    return pl.pallas_call(
        body,
        grid=(Bv,),
        in_specs=[blk_f, blk_f, blk_w, blk_b, blk_w, blk_b],
        out_specs=[blk_o, blk_o, blk_o, blk_o],
        out_shape=[jax.ShapeDtypeStruct((Bv, N, C), f32)] * 4,
    )(feat1, feat2, Wt11, b11, Wt22, b22)


def _tc_dists(pc1t, pc2p, pc2t, pc1p, pc2ps, pc1ps):
    N = Nv
    NB = 512

    def body(a1_ref, b2_ref, a2_ref, b1_ref, b2s_ref, b1s_ref,
             g12_ref, g21_ref, g12s_ref, g21s_ref):
        for b_ref, a_ref, o_ref in ((b2_ref, a1_ref, g12_ref),
                                    (b1_ref, a2_ref, g21_ref),
                                    (b2s_ref, a1_ref, g12s_ref),
                                    (b1s_ref, a2_ref, g21s_ref)):
            p2 = b_ref[...]
            n2 = jnp.sum(p2 * p2, axis=0, keepdims=True)
            o_ref[...] = n2 - 2.0 * jnp.dot(
                a_ref[...], p2, precision=lax.Precision.HIGHEST,
                preferred_element_type=f32)

    blk_a = pl.BlockSpec((None, NB, 16), lambda b, i: (b, i, 0))
    blk_p = pl.BlockSpec((None, 16, N), lambda b, i: (b, 0, 0))
    blk_o = pl.BlockSpec((None, NB, N), lambda b, i: (b, i, 0))
    return pl.pallas_call(
        body,
        grid=(Bv, N // NB),
        in_specs=[blk_a, blk_p, blk_a, blk_p, blk_p, blk_p],
        out_specs=[blk_o, blk_o, blk_o, blk_o],
        out_shape=[jax.ShapeDtypeStruct((Bv, N, N), f32)] * 4,
    )(pc1t, pc2p, pc2t, pc1p, pc2ps, pc1ps)


def _bf_min(v, iota):
    for d in (8, 4, 2, 1):
        v = jnp.minimum(v, v[iota ^ d])
    return v


def _sc_topk_gather(dist, dist_s, tab, xyzt):
    mesh = plsc.VectorSubcoreMesh(core_axis_name="c", subcore_axis_name="s",
                                  num_cores=2, num_subcores=16)
    CH8 = 8
    NCH8 = RPW // CH8

    @functools.partial(
        pl.kernel,
        out_type=[
            jax.ShapeDtypeStruct((ROWS * NS // 128, 128), i32),
            jax.ShapeDtypeStruct((ROWS * NS, Cv), f32),
            jax.ShapeDtypeStruct((ROWS * NS, 16), f32),
        ],
        mesh=mesh,
        scratch_types=[
            pltpu.VMEM((2, CH8, Nv), f32),
            pltpu.VMEM((2, CH8, Nv), f32),
            pltpu.VMEM((1, 128), i32),
            pltpu.VMEM((CH8 * NS, Cv), f32),
            pltpu.VMEM((CH8 * NS, 16), f32),
            pltpu.SemaphoreType.DMA,
            pltpu.SemaphoreType.DMA,
            pltpu.SemaphoreType.DMA,
        ],
        compiler_params=pltpu.CompilerParams(use_tc_tiling_on_sc=False),
        name="sc_topk_gather",
    )
    def k(dist_h, dists_h, tab_h, xyzt_h, idx_o, feat_o, xyz_o,
          dbuf, sbuf, ibuf, fbuf, xbuf, dsem_a, dsem_b, gsem):
        cid = lax.axis_index("c")
        sid = lax.axis_index("s")
        wid = sid * 2 + cid
        row0 = wid * RPW
        boff = (wid // 16) * Nv
        iota = lax.iota(i32, 16)
        inf16 = lax.broadcast(jnp.float32(jnp.inf), (16,))

        def issue(ci, slot, sem):
            pltpu.async_copy(
                dist_h.at[pl.ds(row0 + ci * CH8, CH8)], dbuf.at[slot], sem)
            pltpu.async_copy(
                dists_h.at[pl.ds(row0 + ci * CH8, CH8)], sbuf.at[slot], sem)

        def wait_slot(ci, slot, sem):
            pltpu.make_async_copy(
                dist_h.at[pl.ds(row0 + ci * CH8, CH8)], dbuf.at[slot], sem
            ).wait()
            pltpu.make_async_copy(
                dists_h.at[pl.ds(row0 + ci * CH8, CH8)], sbuf.at[slot], sem
            ).wait()

        issue(0, 0, dsem_a)

        def chunk_mins(buf, slot, r):
            ms = []
            for q in range(8):
                m = buf[slot, r, pl.ds(q * 256, 16)]
                for i in range(1, 16):
                    m = jnp.minimum(m, buf[slot, r, pl.ds(q * 256 + i * 16, 16)])
                ms.append(m)
            return ms

        def one_ext(it, a_v, ms, slot, r):
            m = ms[0]
            p = iota * 0
            for q in range(1, 8):
                t = ms[q] < m
                m = jnp.where(t, ms[q], m)
                p = jnp.where(t, q, p)
            vstar = _bf_min(m, iota)
            lstar = _bf_min(jnp.where(m == vstar, iota, 16), iota)
            qstar = p[lstar]
            cvec = qstar * 16 + lstar
            cstar = cvec[0]
            V = dbuf[slot, r, pl.ds(cstar * 16, 16)]
            istar = _bf_min(jnp.where(V == vstar, iota, 16), iota)
            a_v = jnp.where(iota == it, cvec * 16 + istar, a_v)
            Vm = jnp.where(iota == istar, inf16, V)
            dbuf[slot, r, pl.ds(cstar * 16, 16)] = Vm
            newmin = _bf_min(Vm, iota)
            cond_l = iota == lstar
            for q in range(8):
                ms[q] = jnp.where((qstar == q) & cond_l, newmin, ms[q])
            return a_v, ms

        def topk_row2(j, slot):
            rA = j * 2
            rB = rA + 1
            msA = chunk_mins(sbuf, slot, rA)
            msB = chunk_mins(sbuf, slot, rB)

            def ext(it, carry):
                avA, avB = carry[0], carry[1]
                msA = list(carry[2:10])
                msB = list(carry[10:18])
                avA, msA = one_ext(it, avA, msA, slot, rA)
                avB, msB = one_ext(it, avB, msB, slot, rB)
                return tuple([avA, avB] + msA + msB)

            out = lax.fori_loop(0, 16, ext,
                                tuple([iota * 0, iota * 0] + msA + msB))
            ibuf[0, pl.ds(rA * 16, 16)] = out[0] + boff
            ibuf[0, pl.ds(rB * 16, 16)] = out[1] + boff
            return slot

        def finish_chunk(ci, slot):
            lax.fori_loop(0, CH8 // 2, topk_row2, slot)
            h0 = pltpu.async_copy(tab_h.at[ibuf.at[0]], fbuf, gsem)
            h2 = pltpu.async_copy(xyzt_h.at[ibuf.at[0]], xbuf, gsem)
            h0.wait()
            h2.wait()
            out0 = (row0 + ci * CH8) * NS
            pltpu.sync_copy(fbuf, feat_o.at[pl.ds(out0, CH8 * NS)])
            pltpu.sync_copy(xbuf, xyz_o.at[pl.ds(out0, CH8 * NS)])
            pltpu.sync_copy(ibuf, idx_o.at[pl.ds(wid * NCH8 + ci, 1)])

        def pair(t, _):
            ca = 2 * t
            cb = ca + 1
            issue(cb, 1, dsem_b)
            wait_slot(ca, 0, dsem_a)
            finish_chunk(ca, 0)
            pl.when(t + 1 < NCH8 // 2)(lambda: issue(ca + 2, 0, dsem_a))
            wait_slot(cb, 1, dsem_b)
            finish_chunk(cb, 1)
            return 0

        lax.fori_loop(0, NCH8 // 2, pair, 0)

    return k(dist, dist_s, tab, xyzt)


def _sc_gather(idx, tab):
    mesh = plsc.VectorSubcoreMesh(core_axis_name="c", subcore_axis_name="s",
                                  num_cores=2, num_subcores=16)
    NR = idx.shape[0] // NW

    @functools.partial(
        pl.kernel,
        out_type=jax.ShapeDtypeStruct((ROWS * NS, Cv), f32),
        mesh=mesh,
        scratch_types=[
            pltpu.VMEM((1, 128), i32),
            pltpu.VMEM((128, Cv), f32),
            pltpu.SemaphoreType.DMA,
        ],
        compiler_params=pltpu.CompilerParams(use_tc_tiling_on_sc=False),
        name="sc_gather_only",
    )
    def k2(idx_h, tab_h, feat_o, ibuf, fbuf, gsem):
        cid = lax.axis_index("c")
        sid = lax.axis_index("s")
        wid = sid * 2 + cid

        def body(ci, _):
            rrow = wid * NR + ci
            pltpu.sync_copy(idx_h.at[pl.ds(rrow, 1)], ibuf)
            pltpu.async_copy(tab_h.at[ibuf.at[0]], fbuf, gsem).wait()
            pltpu.sync_copy(fbuf, feat_o.at[pl.ds(rrow * 128, 128)])
            return 0

        lax.fori_loop(0, NR, body, 0)

    return k2(idx, tab)


def _tc_dense(gf_flat, gx_flat, p1t, pc1t, W_pos, b_pos, gamma0, beta0,
              mlps, final):
    N, C = Nv, Cv
    M4 = N * NS // 4
    C4 = 4 * C

    gf = gf_flat.reshape(Bv, M4, C4)
    gx = gx_flat.reshape(Bv, M4, 64)
    eye4 = jnp.eye(4, dtype=f32)
    posW4 = jnp.kron(eye4, jnp.pad(W_pos.T, ((0, 13), (0, 0))))
    posb4 = jnp.tile(b_pos, 4)
    g0_4 = jnp.tile(gamma0, 4)
    b0_4 = jnp.tile(beta0, 4)
    ch = (jnp.arange(C4) % C) // (C // G)
    mmat = (ch[:, None] == ch[None, :]).astype(f32)
    mlp_packed = []
    for (W, b, gg, bb) in mlps:
        mlp_packed.append((jnp.kron(eye4, W.T), jnp.tile(b, 4),
                           jnp.tile(gg, 4), jnp.tile(bb, 4)))
    n_mlp = len(mlps)
    cnt = float(M4 * 64)

    def gn_leaky(x, a_g, a_b, mm):
        s1 = jnp.sum(x, axis=0, keepdims=True)
        s2 = jnp.sum(x * x, axis=0, keepdims=True)
        mean = jnp.dot(s1, mm, preferred_element_type=f32) / cnt
        ex2 = jnp.dot(s2, mm, preferred_element_type=f32) / cnt
        var = ex2 - mean * mean
        a = a_g[None, :] * lax.rsqrt(var + 1e-5)
        d = a_b[None, :] - mean * a
        y = x * a + d
        return jnp.maximum(y, 0.1 * y)

    def body(*refs):
        (gf_ref, gx_ref, p1t4_ref, pc1t4_ref, posw_ref, posb_ref,
         g0_ref, b0_ref, mm_ref) = refs[:9]
        pos = 9
        mlp_refs = refs[pos:pos + 4 * n_mlp]
        pos += 4 * n_mlp
        if final is not None:
            wf_ref, bf_ref = refs[pos:pos + 2]
            pos += 2
        out_ref, outt_ref = refs[pos:pos + 2]

        mm = mm_ref[...]
        p1t4 = jnp.concatenate([p1t4_ref[...]] * 4, axis=-1)
        pc1t4 = jnp.concatenate([pc1t4_ref[...]] * 4, axis=-1)
        dx3 = gx_ref[...].reshape(N, 4, 64) - pc1t4[:, None, :]
        p = jnp.dot(dx3.reshape(M4, 64), posw_ref[...],
                    preferred_element_type=f32) + posb_ref[...][None, :]
        x3 = (gf_ref[...] + p).reshape(N, 4, C4) + p1t4[:, None, :]
        h = gn_leaky(x3.reshape(M4, C4), g0_ref[...], b0_ref[...], mm)
        for li in range(n_mlp):
            w4_ref, b4_ref, gg_ref, bb_ref = mlp_refs[4 * li:4 * li + 4]
            hc = jnp.dot(h, w4_ref[...],
                         preferred_element_type=f32) + b4_ref[...][None, :]
            h = gn_leaky(hc, gg_ref[...], bb_ref[...], mm)
        m = jnp.maximum(jnp.maximum(h[:, 0:C], h[:, C:2 * C]),
                        jnp.maximum(h[:, 2 * C:3 * C], h[:, 3 * C:4 * C]))
        hm = jnp.max(m.reshape(N, 4, C), axis=1)
        if final is not None:
            ot = jnp.dot(hm, wf_ref[...],
                         preferred_element_type=f32) + bf_ref[...][None, :]
        else:
            ot = hm
        outt_ref[...] = ot
        out_ref[...] = ot.T

    inputs = [gf, gx, p1t, pc1t, posW4, posb4, g0_4, b0_4, mmat]
    in_specs = [
        pl.BlockSpec((None, M4, C4), lambda b: (b, 0, 0)),
        pl.BlockSpec((None, M4, 64), lambda b: (b, 0, 0)),
        pl.BlockSpec((None, N, Cv), lambda b: (b, 0, 0)),
        pl.BlockSpec((None, N, 16), lambda b: (b, 0, 0)),
        pl.BlockSpec((64, C4), lambda b: (0, 0)),
        pl.BlockSpec((C4,), lambda b: (0,)),
        pl.BlockSpec((C4,), lambda b: (0,)),
        pl.BlockSpec((C4,), lambda b: (0,)),
        pl.BlockSpec((C4, C4), lambda b: (0, 0)),
    ]
    for tup in mlp_packed:
        inputs.extend(tup)
        in_specs.extend([
            pl.BlockSpec((C4, C4), lambda b: (0, 0)),
            pl.BlockSpec((C4,), lambda b: (0,)),
            pl.BlockSpec((C4,), lambda b: (0,)),
            pl.BlockSpec((C4,), lambda b: (0,)),
        ])
    if final is not None:
        W_f, b_f = final
        inputs.extend([W_f.T, b_f])
        in_specs.extend([
            pl.BlockSpec((C, C), lambda b: (0, 0)),
            pl.BlockSpec((C,), lambda b: (0,)),
        ])

    out, outt = pl.pallas_call(
        body,
        grid=(Bv,),
        in_specs=in_specs,
        out_specs=[
            pl.BlockSpec((None, C, N), lambda b: (b, 0, 0)),
            pl.BlockSpec((None, N, C), lambda b: (b, 0, 0)),
        ],
        out_shape=[
            jax.ShapeDtypeStruct((Bv, C, N), f32),
            jax.ShapeDtypeStruct((Bv, N, C), f32),
        ],
    )(*inputs)
    return out, outt


def kernel(pc1, pc2, feat1, feat2, W_t11, b_t11, W_t22, b_t22, W_pos1, b_pos1, gamma_bn1, beta_bn1, Wm1_1, bm1_1, gm1_1, btm1_1, Wm1_2, bm1_2, gm1_2, btm1_2, W_t1, b_t1, W_t2, b_t2, W_pos2, b_pos2, gamma_bn2, beta_bn2, Wm2_1, bm2_1, gm2_1, btm2_1):
    N, C = Nv, Cv

    pc1t = jnp.pad(jnp.transpose(pc1, (0, 2, 1)), ((0, 0), (0, 0), (0, 13)))
    pc2t = jnp.pad(jnp.transpose(pc2, (0, 2, 1)), ((0, 0), (0, 0), (0, 13)))
    pc1p = jnp.pad(pc1, ((0, 0), (0, 13), (0, 0)))
    pc2p = jnp.pad(pc2, ((0, 0), (0, 13), (0, 0)))

    pp = jnp.arange(N)
    perm = (pp // 256) * 256 + (pp % 16) * 16 + (pp % 256) // 16
    pc2ps = pc2p[:, :, perm]
    pc1ps = pc1p[:, :, perm]

    p1t_1, p1t_2, f1bt3, f2t3 = _tc_transforms(
        feat1, feat2, W_t11.T, b_t11, W_t22.T, b_t22)
    G12, G21, G12s, G21s = _tc_dists(pc1t, pc2p, pc2t, pc1p, pc2ps, pc1ps)

    f2t = f2t3.reshape(Bv * N, C)
    f1bt = f1bt3.reshape(Bv * N, C)

    idx12, gf1, gx1 = _sc_topk_gather(G12.reshape(Bv * N, N),
                                      G12s.reshape(Bv * N, N), f2t,
                                      pc2t.reshape(Bv * N, 16))
    idx21, gf2, gx2 = _sc_topk_gather(G21.reshape(Bv * N, N),
                                      G21s.reshape(Bv * N, N), f1bt,
                                      pc1t.reshape(Bv * N, 16))

    mlp1 = [(Wm1_1, bm1_1, gm1_1, btm1_1), (Wm1_2, bm1_2, gm1_2, btm1_2)]
    feat1_new, feat1_new_t = _tc_dense(
        gf1, gx1, p1t_1, pc1t, W_pos1, b_pos1, gamma_bn1, beta_bn1, mlp1,
        (W_t1, b_t1))
    feat2_new, feat2_new_t = _tc_dense(
        gf2, gx2, p1t_2, pc2t, W_pos1, b_pos1, gamma_bn1, beta_bn1, mlp1,
        (W_t2, b_t2))

    gf3 = _sc_gather(idx12, feat2_new_t.reshape(Bv * N, C))
    mlp2 = [(Wm2_1, bm2_1, gm2_1, btm2_1)]
    feat1_final, _ = _tc_dense(
        gf3, gx1, feat1_new_t, pc1t, W_pos2, b_pos2, gamma_bn2, beta_bn2,
        mlp2, None)

    return (feat1_new, feat2_new, feat1_final)


# speedup vs baseline: 12.1518x; 1.0162x over previous
import functools

import jax
import jax.numpy as jnp
from jax import lax
from jax.experimental import pallas as pl
from jax.experimental.pallas import tpu as pltpu
from jax.experimental.pallas import tpu_sc as plsc

Bv, Nv, Cv = 2, 2048, 64
NS = 16
G = 4
f32 = jnp.float32
i32 = jnp.int32

NW = 32
ROWS = Bv * Nv
RPW = ROWS // NW
CH = 16
NCH = RPW // CH


def _tc_transforms(feat1, feat2, Wt11, b11, Wt22, b22, pc1p, pc2p):
    N, C = Nv, Cv

    def body(f1_ref, f2_ref, w11_ref, b11_ref, w22_ref, b22_ref,
             p1_ref, p2_ref, e_ref, f1t_ref, f2bt_ref, f1bt_ref, f2t_ref,
             p1t_ref, p2t_ref):
        dn = (((0,), (0,)), ((), ()))
        e = e_ref[...]
        p1t_ref[...] = lax.dot_general(
            p1_ref[...], e, dn, precision=lax.Precision.HIGHEST,
            preferred_element_type=f32)
        p2t_ref[...] = lax.dot_general(
            p2_ref[...], e, dn, precision=lax.Precision.HIGHEST,
            preferred_element_type=f32)
        for w_ref, b_ref, o1_ref, o2_ref in (
                (w11_ref, b11_ref, f1t_ref, f2bt_ref),
                (w22_ref, b22_ref, f1bt_ref, f2t_ref)):
            w = w_ref[...]
            b = b_ref[...][None, :]
            o1_ref[...] = lax.dot_general(
                f1_ref[...], w, dn, preferred_element_type=f32) + b
            o2_ref[...] = lax.dot_general(
                f2_ref[...], w, dn, preferred_element_type=f32) + b

    blk_f = pl.BlockSpec((None, C, N), lambda b: (b, 0, 0))
    blk_w = pl.BlockSpec((C, C), lambda b: (0, 0))
    blk_b = pl.BlockSpec((C,), lambda b: (0,))
    blk_o = pl.BlockSpec((None, N, C), lambda b: (b, 0, 0))
    blk_p = pl.BlockSpec((None, 16, N), lambda b: (b, 0, 0))
    blk_e = pl.BlockSpec((16, 16), lambda b: (0, 0))
    blk_t = pl.BlockSpec((None, N, 16), lambda b: (b, 0, 0))
    eye16 = jnp.eye(16, dtype=f32)
    return pl.pallas_call(
        body,
        grid=(Bv,),
        in_specs=[blk_f, blk_f, blk_w, blk_b, blk_w, blk_b, blk_p, blk_p,
                  blk_e],
        out_specs=[blk_o, blk_o, blk_o, blk_o, blk_t, blk_t],
        out_shape=[jax.ShapeDtypeStruct((Bv, N, C), f32)] * 4
        + [jax.ShapeDtypeStruct((Bv, N, 16), f32)] * 2,
    )(feat1, feat2, Wt11, b11, Wt22, b22, pc1p, pc2p, eye16)


def _tc_dists(pc1t, pc2p, pc2t, pc1p, pc2ps, pc1ps):
    N = Nv
    NB = 512

    def body(a1_ref, b2_ref, a2_ref, b1_ref, b2s_ref, b1s_ref,
             g12_ref, g21_ref, g12s_ref, g21s_ref):
        for b_ref, a_ref, o_ref in ((b2_ref, a1_ref, g12_ref),
                                    (b1_ref, a2_ref, g21_ref),
                                    (b2s_ref, a1_ref, g12s_ref),
                                    (b1s_ref, a2_ref, g21s_ref)):
            p2 = b_ref[...]
            n2 = jnp.sum(p2 * p2, axis=0, keepdims=True)
            o_ref[...] = n2 - 2.0 * jnp.dot(
                a_ref[...], p2, precision=lax.Precision.HIGHEST,
                preferred_element_type=f32)

    blk_a = pl.BlockSpec((None, NB, 16), lambda b, i: (b, i, 0))
    blk_p = pl.BlockSpec((None, 16, N), lambda b, i: (b, 0, 0))
    blk_o = pl.BlockSpec((None, NB, N), lambda b, i: (b, i, 0))
    return pl.pallas_call(
        body,
        grid=(Bv, N // NB),
        in_specs=[blk_a, blk_p, blk_a, blk_p, blk_p, blk_p],
        out_specs=[blk_o, blk_o, blk_o, blk_o],
        out_shape=[jax.ShapeDtypeStruct((Bv, N, N), f32)] * 4,
    )(pc1t, pc2p, pc2t, pc1p, pc2ps, pc1ps)


def _bf_min(v, iota):
    for d in (8, 4, 2, 1):
        v = jnp.minimum(v, v[iota ^ d])
    return v


def _sc_topk_gather(dist, dist_s, tab, xyzt):
    mesh = plsc.VectorSubcoreMesh(core_axis_name="c", subcore_axis_name="s",
                                  num_cores=2, num_subcores=16)
    CH8 = 8
    NCH8 = RPW // CH8

    @functools.partial(
        pl.kernel,
        out_type=[
            jax.ShapeDtypeStruct((ROWS * NS // 128, 128), i32),
            jax.ShapeDtypeStruct((ROWS * NS, Cv), f32),
            jax.ShapeDtypeStruct((ROWS * NS, 16), f32),
        ],
        mesh=mesh,
        scratch_types=[
            pltpu.VMEM((2, CH8, Nv), f32),
            pltpu.VMEM((2, CH8, Nv), f32),
            pltpu.VMEM((1, 128), i32),
            pltpu.VMEM((CH8 * NS, Cv), f32),
            pltpu.VMEM((CH8 * NS, 16), f32),
            pltpu.SemaphoreType.DMA,
            pltpu.SemaphoreType.DMA,
            pltpu.SemaphoreType.DMA,
        ],
        compiler_params=pltpu.CompilerParams(use_tc_tiling_on_sc=False),
        name="sc_topk_gather",
    )
    def k(dist_h, dists_h, tab_h, xyzt_h, idx_o, feat_o, xyz_o,
          dbuf, sbuf, ibuf, fbuf, xbuf, dsem_a, dsem_b, gsem):
        cid = lax.axis_index("c")
        sid = lax.axis_index("s")
        wid = sid * 2 + cid
        row0 = wid * RPW
        boff = (wid // 16) * Nv
        iota = lax.iota(i32, 16)
        inf16 = lax.broadcast(jnp.float32(jnp.inf), (16,))

        def issue(ci, slot, sem):
            pltpu.async_copy(
                dist_h.at[pl.ds(row0 + ci * CH8, CH8)], dbuf.at[slot], sem)
            pltpu.async_copy(
                dists_h.at[pl.ds(row0 + ci * CH8, CH8)], sbuf.at[slot], sem)

        def wait_slot(ci, slot, sem):
            pltpu.make_async_copy(
                dist_h.at[pl.ds(row0 + ci * CH8, CH8)], dbuf.at[slot], sem
            ).wait()
            pltpu.make_async_copy(
                dists_h.at[pl.ds(row0 + ci * CH8, CH8)], sbuf.at[slot], sem
            ).wait()

        issue(0, 0, dsem_a)

        def chunk_mins(buf, slot, r):
            ms = []
            for q in range(8):
                m = buf[slot, r, pl.ds(q * 256, 16)]
                for i in range(1, 16):
                    m = jnp.minimum(m, buf[slot, r, pl.ds(q * 256 + i * 16, 16)])
                ms.append(m)
            return ms

        def one_ext(it, a_v, ms, slot, r):
            m = ms[0]
            p = iota * 0
            for q in range(1, 8):
                t = ms[q] < m
                m = jnp.where(t, ms[q], m)
                p = jnp.where(t, q, p)
            vstar = _bf_min(m, iota)
            lstar = _bf_min(jnp.where(m == vstar, iota, 16), iota)
            qstar = p[lstar]
            cvec = qstar * 16 + lstar
            cstar = cvec[0]
            V = dbuf[slot, r, pl.ds(cstar * 16, 16)]
            istar = _bf_min(jnp.where(V == vstar, iota, 16), iota)
            a_v = jnp.where(iota == it, cvec * 16 + istar, a_v)
            Vm = jnp.where(iota == istar, inf16, V)
            dbuf[slot, r, pl.ds(cstar * 16, 16)] = Vm
            newmin = _bf_min(Vm, iota)
            cond_l = iota == lstar
            for q in range(8):
                ms[q] = jnp.where((qstar == q) & cond_l, newmin, ms[q])
            return a_v, ms

        def topk_row2(j, slot):
            rA = j * 2
            rB = rA + 1
            msA = chunk_mins(sbuf, slot, rA)
            msB = chunk_mins(sbuf, slot, rB)

            def ext(it, carry):
                avA, avB = carry[0], carry[1]
                msA = list(carry[2:10])
                msB = list(carry[10:18])
                avA, msA = one_ext(it, avA, msA, slot, rA)
                avB, msB = one_ext(it, avB, msB, slot, rB)
                return tuple([avA, avB] + msA + msB)

            out = lax.fori_loop(0, 16, ext,
                                tuple([iota * 0, iota * 0] + msA + msB))
            ibuf[0, pl.ds(rA * 16, 16)] = out[0] + boff
            ibuf[0, pl.ds(rB * 16, 16)] = out[1] + boff
            return slot

        def finish_chunk(ci, slot):
            lax.fori_loop(0, CH8 // 2, topk_row2, slot)
            h0 = pltpu.async_copy(tab_h.at[ibuf.at[0]], fbuf, gsem)
            h2 = pltpu.async_copy(xyzt_h.at[ibuf.at[0]], xbuf, gsem)
            h0.wait()
            h2.wait()
            out0 = (row0 + ci * CH8) * NS
            pltpu.sync_copy(fbuf, feat_o.at[pl.ds(out0, CH8 * NS)])
            pltpu.sync_copy(xbuf, xyz_o.at[pl.ds(out0, CH8 * NS)])
            pltpu.sync_copy(ibuf, idx_o.at[pl.ds(wid * NCH8 + ci, 1)])

        def pair(t, _):
            ca = 2 * t
            cb = ca + 1
            issue(cb, 1, dsem_b)
            wait_slot(ca, 0, dsem_a)
            finish_chunk(ca, 0)
            pl.when(t + 1 < NCH8 // 2)(lambda: issue(ca + 2, 0, dsem_a))
            wait_slot(cb, 1, dsem_b)
            finish_chunk(cb, 1)
            return 0

        lax.fori_loop(0, NCH8 // 2, pair, 0)

    return k(dist, dist_s, tab, xyzt)


def _sc_gather(idx, tab):
    mesh = plsc.VectorSubcoreMesh(core_axis_name="c", subcore_axis_name="s",
                                  num_cores=2, num_subcores=16)
    NR = idx.shape[0] // NW

    @functools.partial(
        pl.kernel,
        out_type=jax.ShapeDtypeStruct((ROWS * NS, Cv), f32),
        mesh=mesh,
        scratch_types=[
            pltpu.VMEM((1, 128), i32),
            pltpu.VMEM((128, Cv), f32),
            pltpu.SemaphoreType.DMA,
        ],
        compiler_params=pltpu.CompilerParams(use_tc_tiling_on_sc=False),
        name="sc_gather_only",
    )
    def k2(idx_h, tab_h, feat_o, ibuf, fbuf, gsem):
        cid = lax.axis_index("c")
        sid = lax.axis_index("s")
        wid = sid * 2 + cid

        def body(ci, _):
            rrow = wid * NR + ci
            pltpu.sync_copy(idx_h.at[pl.ds(rrow, 1)], ibuf)
            pltpu.async_copy(tab_h.at[ibuf.at[0]], fbuf, gsem).wait()
            pltpu.sync_copy(fbuf, feat_o.at[pl.ds(rrow * 128, 128)])
            return 0

        lax.fori_loop(0, NR, body, 0)

    return k2(idx, tab)


def _tc_dense(gf_flat, gx_flat, p1t, pc1t, W_pos, b_pos, gamma0, beta0,
              mlps, final):
    N, C = Nv, Cv
    M4 = N * NS // 4
    C4 = 4 * C

    gf = gf_flat.reshape(Bv, M4, C4)
    gx = gx_flat.reshape(Bv, M4, 64)
    eye4 = jnp.eye(4, dtype=f32)
    posW4 = jnp.kron(eye4, jnp.pad(W_pos.T, ((0, 13), (0, 0))))
    posb4 = jnp.tile(b_pos, 4)
    g0_4 = jnp.tile(gamma0, 4)
    b0_4 = jnp.tile(beta0, 4)
    ch = (jnp.arange(C4) % C) // (C // G)
    mmat = (ch[:, None] == ch[None, :]).astype(f32)
    mlp_packed = []
    for (W, b, gg, bb) in mlps:
        mlp_packed.append((jnp.kron(eye4, W.T), jnp.tile(b, 4),
                           jnp.tile(gg, 4), jnp.tile(bb, 4)))
    n_mlp = len(mlps)
    cnt = float(M4 * 64)

    def gn_leaky(x, a_g, a_b, mm):
        s1 = jnp.sum(x, axis=0, keepdims=True)
        s2 = jnp.sum(x * x, axis=0, keepdims=True)
        mean = jnp.dot(s1, mm, preferred_element_type=f32) / cnt
        ex2 = jnp.dot(s2, mm, preferred_element_type=f32) / cnt
        var = ex2 - mean * mean
        a = a_g[None, :] * lax.rsqrt(var + 1e-5)
        d = a_b[None, :] - mean * a
        y = x * a + d
        return jnp.maximum(y, 0.1 * y)

    def body(*refs):
        (gf_ref, gx_ref, p1t4_ref, pc1t4_ref, posw_ref, posb_ref,
         g0_ref, b0_ref, mm_ref) = refs[:9]
        pos = 9
        mlp_refs = refs[pos:pos + 4 * n_mlp]
        pos += 4 * n_mlp
        if final is not None:
            wf_ref, bf_ref = refs[pos:pos + 2]
            pos += 2
        out_ref, outt_ref = refs[pos:pos + 2]

        mm = mm_ref[...]
        p1t4 = jnp.concatenate([p1t4_ref[...]] * 4, axis=-1)
        pc1t4 = jnp.concatenate([pc1t4_ref[...]] * 4, axis=-1)
        dx3 = gx_ref[...].reshape(N, 4, 64) - pc1t4[:, None, :]
        p = jnp.dot(dx3.reshape(M4, 64), posw_ref[...],
                    preferred_element_type=f32) + posb_ref[...][None, :]
        x3 = (gf_ref[...] + p).reshape(N, 4, C4) + p1t4[:, None, :]
        h = gn_leaky(x3.reshape(M4, C4), g0_ref[...], b0_ref[...], mm)
        for li in range(n_mlp):
            w4_ref, b4_ref, gg_ref, bb_ref = mlp_refs[4 * li:4 * li + 4]
            hc = jnp.dot(h, w4_ref[...],
                         preferred_element_type=f32) + b4_ref[...][None, :]
            h = gn_leaky(hc, gg_ref[...], bb_ref[...], mm)
        m = jnp.maximum(jnp.maximum(h[:, 0:C], h[:, C:2 * C]),
                        jnp.maximum(h[:, 2 * C:3 * C], h[:, 3 * C:4 * C]))
        hm = jnp.max(m.reshape(N, 4, C), axis=1)
        if final is not None:
            ot = jnp.dot(hm, wf_ref[...],
                         preferred_element_type=f32) + bf_ref[...][None, :]
        else:
            ot = hm
        outt_ref[...] = ot
        out_ref[...] = ot.T

    inputs = [gf, gx, p1t, pc1t, posW4, posb4, g0_4, b0_4, mmat]
    in_specs = [
        pl.BlockSpec((None, M4, C4), lambda b: (b, 0, 0)),
        pl.BlockSpec((None, M4, 64), lambda b: (b, 0, 0)),
        pl.BlockSpec((None, N, Cv), lambda b: (b, 0, 0)),
        pl.BlockSpec((None, N, 16), lambda b: (b, 0, 0)),
        pl.BlockSpec((64, C4), lambda b: (0, 0)),
        pl.BlockSpec((C4,), lambda b: (0,)),
        pl.BlockSpec((C4,), lambda b: (0,)),
        pl.BlockSpec((C4,), lambda b: (0,)),
        pl.BlockSpec((C4, C4), lambda b: (0, 0)),
    ]
    for tup in mlp_packed:
        inputs.extend(tup)
        in_specs.extend([
            pl.BlockSpec((C4, C4), lambda b: (0, 0)),
            pl.BlockSpec((C4,), lambda b: (0,)),
            pl.BlockSpec((C4,), lambda b: (0,)),
            pl.BlockSpec((C4,), lambda b: (0,)),
        ])
    if final is not None:
        W_f, b_f = final
        inputs.extend([W_f.T, b_f])
        in_specs.extend([
            pl.BlockSpec((C, C), lambda b: (0, 0)),
            pl.BlockSpec((C,), lambda b: (0,)),
        ])

    out, outt = pl.pallas_call(
        body,
        grid=(Bv,),
        in_specs=in_specs,
        out_specs=[
            pl.BlockSpec((None, C, N), lambda b: (b, 0, 0)),
            pl.BlockSpec((None, N, C), lambda b: (b, 0, 0)),
        ],
        out_shape=[
            jax.ShapeDtypeStruct((Bv, C, N), f32),
            jax.ShapeDtypeStruct((Bv, N, C), f32),
        ],
    )(*inputs)
    return out, outt


def kernel(pc1, pc2, feat1, feat2, W_t11, b_t11, W_t22, b_t22, W_pos1, b_pos1, gamma_bn1, beta_bn1, Wm1_1, bm1_1, gm1_1, btm1_1, Wm1_2, bm1_2, gm1_2, btm1_2, W_t1, b_t1, W_t2, b_t2, W_pos2, b_pos2, gamma_bn2, beta_bn2, Wm2_1, bm2_1, gm2_1, btm2_1):
    N, C = Nv, Cv

    pc1p = jnp.pad(pc1, ((0, 0), (0, 13), (0, 0)))
    pc2p = jnp.pad(pc2, ((0, 0), (0, 13), (0, 0)))

    def shuf(x):
        return x.reshape(Bv, 16, 8, 16, 16).swapaxes(3, 4).reshape(Bv, 16, N)

    pc2ps = shuf(pc2p)
    pc1ps = shuf(pc1p)

    p1t_1, p1t_2, f1bt3, f2t3, pc1t, pc2t = _tc_transforms(
        feat1, feat2, W_t11.T, b_t11, W_t22.T, b_t22, pc1p, pc2p)
    G12, G21, G12s, G21s = _tc_dists(pc1t, pc2p, pc2t, pc1p, pc2ps, pc1ps)

    f2t = f2t3.reshape(Bv * N, C)
    f1bt = f1bt3.reshape(Bv * N, C)

    idx12, gf1, gx1 = _sc_topk_gather(G12.reshape(Bv * N, N),
                                      G12s.reshape(Bv * N, N), f2t,
                                      pc2t.reshape(Bv * N, 16))
    idx21, gf2, gx2 = _sc_topk_gather(G21.reshape(Bv * N, N),
                                      G21s.reshape(Bv * N, N), f1bt,
                                      pc1t.reshape(Bv * N, 16))

    mlp1 = [(Wm1_1, bm1_1, gm1_1, btm1_1), (Wm1_2, bm1_2, gm1_2, btm1_2)]
    feat1_new, feat1_new_t = _tc_dense(
        gf1, gx1, p1t_1, pc1t, W_pos1, b_pos1, gamma_bn1, beta_bn1, mlp1,
        (W_t1, b_t1))
    feat2_new, feat2_new_t = _tc_dense(
        gf2, gx2, p1t_2, pc2t, W_pos1, b_pos1, gamma_bn1, beta_bn1, mlp1,
        (W_t2, b_t2))

    gf3 = _sc_gather(idx12, feat2_new_t.reshape(Bv * N, C))
    mlp2 = [(Wm2_1, bm2_1, gm2_1, btm2_1)]
    feat1_final, _ = _tc_dense(
        gf3, gx1, feat1_new_t, pc1t, W_pos2, b_pos2, gamma_bn2, beta_bn2,
        mlp2, None)

    return (feat1_new, feat2_new, feat1_final)
